# Initial kernel scaffold; baseline (speedup 1.0000x reference)
import jax
import jax.numpy as jnp
from jax.experimental import pallas as pl


def kernel(x, enc_w0, enc_b0, enc_w1, enc_b1, f1_wr, f1_br, f1_we, f1_be, f1_wg, f1_bg, f2_wr, f2_br, f2_we, f2_be, f2_wg, f2_bg, aux_w0, aux_b0, aux_w1, aux_b1, ccc_w1, ccc_b1, ccc_w2, ccc_b2, gva_wq, gva_wk, gva_wv, up_w0, up_b0, up_w1, up_b1, cls_w0, cls_b0, cls_w1, cls_b1):
    raise NotImplementedError("write your pallas kernel here")


# speedup vs baseline: 1.8465x; 1.8465x over previous
import functools

import jax
import jax.numpy as jnp
from jax.experimental import pallas as pl

K_NN = 32


def _fps_body(m, c_ref, o_ref):
    cval = c_ref[0]
    n = cval.shape[1]
    iota = jax.lax.broadcasted_iota(jnp.int32, (1, n), 1)
    iota_m = jax.lax.broadcasted_iota(jnp.int32, (1, m), 1)

    def body(i, carry):
        dist, far, idxs = carry
        idxs = jnp.where(iota_m == i, far, idxs)
        cent = jnp.sum(jnp.where(iota == far, cval, 0.0),
                       axis=1, keepdims=True)
        diff = cval - cent
        d = jnp.sum(diff * diff, axis=0, keepdims=True)
        dist = jnp.minimum(dist, d)
        mx = jnp.max(dist)
        sel = jnp.where(dist == mx, iota, n)
        far2 = jnp.min(sel).astype(jnp.int32)
        return dist, far2, idxs

    dist0 = jnp.full((1, n), 1e10, jnp.float32)
    idxs0 = jnp.zeros((1, m), jnp.int32)
    _, _, idxs = jax.lax.fori_loop(0, m, body, (dist0, jnp.int32(0), idxs0))
    o_ref[0] = idxs


def _fps_pallas(coords, m):
    B, N, _ = coords.shape
    cpad = jnp.concatenate(
        [coords.transpose(0, 2, 1),
         jnp.zeros((B, 5, N), jnp.float32)], axis=1)
    out = pl.pallas_call(
        functools.partial(_fps_body, m),
        grid=(B,),
        in_specs=[pl.BlockSpec((1, 8, N), lambda b: (b, 0, 0))],
        out_specs=pl.BlockSpec((1, 1, m), lambda b: (b, 0, 0)),
        out_shape=jax.ShapeDtypeStruct((B, 1, m), jnp.int32),
    )(cpad)
    return out[:, 0, :]


def _local_agg(coords, feats, cidx, wr, br, we, be, wg, bg, rmin, rmax):
    centers = jnp.take_along_axis(coords, cidx[..., None], axis=1)
    cfeat = jnp.take_along_axis(feats, cidx[..., None], axis=1)
    d2 = jnp.sum((centers[:, :, None, :] - coords[:, None, :, :]) ** 2, axis=-1)
    neg_d2, knn_idx = jax.lax.top_k(-d2, K_NN)
    knn_d2 = -neg_d2
    radius = rmin + jax.nn.sigmoid(cfeat @ wr + br) * (rmax - rmin)
    mask = knn_d2 <= radius ** 2
    nfeat = jax.vmap(lambda f, i: f[i])(feats, knn_idx)
    npos = jax.vmap(lambda c, i: c[i])(coords, knn_idx)
    rel = npos - centers[:, :, None, :]
    edge = jnp.concatenate([nfeat, rel], axis=-1)
    e = jax.nn.relu(edge @ we + be)
    e = jnp.where(mask[..., None], e, -1e9)
    pooled = jnp.max(e, axis=2)
    gate = jax.nn.sigmoid(pooled @ wg + bg)
    return pooled * gate


def _channel_ccc(h, w1, b1, w2, b2):
    desc = jnp.mean(h, axis=1)
    a = jax.nn.sigmoid(jax.nn.relu(desc @ w1 + b1) @ w2 + b2)
    return h * a[:, None, :]


def _linear_gva(h, wq, wk, wv):
    q = h @ wq
    k = h @ wk
    v = h @ wv
    attn = jax.nn.softmax(k, axis=1)
    g = jnp.sum(attn * v, axis=1, keepdims=True)
    return h + jax.nn.sigmoid(q) * g


def _nearest_up(h, n):
    m = h.shape[1]
    idx = (jnp.arange(n) * m) // n
    return h[:, idx, :]


def kernel(x, enc_w0, enc_b0, enc_w1, enc_b1,
           f1_wr, f1_br, f1_we, f1_be, f1_wg, f1_bg,
           f2_wr, f2_br, f2_we, f2_be, f2_wg, f2_bg,
           aux_w0, aux_b0, aux_w1, aux_b1,
           ccc_w1, ccc_b1, ccc_w2, ccc_b2,
           gva_wq, gva_wk, gva_wv,
           up_w0, up_b0, up_w1, up_b1,
           cls_w0, cls_b0, cls_w1, cls_b1):
    B, N, _ = x.shape
    coords = x[..., :3]
    m1 = max(1, N // 2)
    m2 = max(1, N // 4)

    idx1 = _fps_pallas(coords, m1)
    idx2 = idx1[:, :m2]

    feats = jax.nn.relu(x @ enc_w0 + enc_b0) @ enc_w1 + enc_b1
    out1 = _local_agg(coords, feats, idx1, f1_wr, f1_br, f1_we, f1_be,
                      f1_wg, f1_bg, 0.02, 0.15)
    out2 = _local_agg(coords, feats, idx2, f2_wr, f2_br, f2_we, f2_be,
                      f2_wg, f2_bg, 0.05, 0.3)
    out1 = _channel_ccc(out1, ccc_w1, ccc_b1, ccc_w2, ccc_b2)
    out2 = _channel_ccc(out2, ccc_w1, ccc_b1, ccc_w2, ccc_b2)
    out1 = _linear_gva(out1, gva_wq, gva_wk, gva_wv)
    out2 = _linear_gva(out2, gva_wq, gva_wk, gva_wv)
    out1_up = _nearest_up(out1, N)
    out2_up = _nearest_up(out2, N)
    fused = jnp.concatenate([out1_up, out2_up], axis=-1)
    fused = jax.nn.relu(fused @ up_w0 + up_b0) @ up_w1 + up_b1
    logits = jax.nn.relu(fused @ cls_w0 + cls_b0) @ cls_w1 + cls_b1
    return logits


# speedup vs baseline: 7.0989x; 3.8444x over previous
import functools

import jax
import jax.numpy as jnp
from jax.experimental import pallas as pl
from jax.experimental.pallas import tpu as pltpu

K_NN = 32


def _fps_body(m, c_ref, o_ref):
    cval = c_ref[0]
    n = cval.shape[1]
    iota = jax.lax.broadcasted_iota(jnp.int32, (1, n), 1)
    iota_m = jax.lax.broadcasted_iota(jnp.int32, (1, m), 1)

    def body(i, carry):
        dist, far, idxs = carry
        idxs = jnp.where(iota_m == i, far, idxs)
        cent = jnp.sum(jnp.where(iota == far, cval, 0.0),
                       axis=1, keepdims=True)
        diff = cval - cent
        d = jnp.sum(diff * diff, axis=0, keepdims=True)
        dist = jnp.minimum(dist, d)
        mx = jnp.max(dist)
        sel = jnp.where(dist == mx, iota, n)
        far2 = jnp.min(sel).astype(jnp.int32)
        return dist, far2, idxs

    dist0 = jnp.full((1, n), 1e10, jnp.float32)
    idxs0 = jnp.zeros((1, m), jnp.int32)
    _, _, idxs = jax.lax.fori_loop(0, m, body, (dist0, jnp.int32(0), idxs0))
    o_ref[0] = idxs


def _fps_pallas(cpad, m):
    B, _, N = cpad.shape
    out = pl.pallas_call(
        functools.partial(_fps_body, m),
        grid=(B,),
        in_specs=[pl.BlockSpec((1, 8, N), lambda b: (b, 0, 0))],
        out_specs=pl.BlockSpec((1, 1, m), lambda b: (b, 0, 0)),
        out_shape=jax.ShapeDtypeStruct((B, 1, m), jnp.int32),
    )(cpad)
    return out[:, 0, :]


def _agg_body(c8_ref, ct_ref, coords_ref, fp_ref, wg_ref, bg_ref,
              out_ref, d2_ref):
    tc = c8_ref.shape[1]
    n = coords_ref.shape[2]
    c8 = c8_ref[0]
    cx, cy, cz = c8[:, 0:1], c8[:, 1:2], c8[:, 2:3]
    r2 = c8[:, 3:4]
    px = coords_ref[0, 0:1, :]
    py = coords_ref[0, 1:2, :]
    pz = coords_ref[0, 2:3, :]
    dx, dy, dz = cx - px, cy - py, cz - pz
    d2_ref[...] = (dx * dx + dy * dy) + dz * dz
    ct = ct_ref[0]
    iota = jax.lax.broadcasted_iota(jnp.int32, (1, n), 1)

    def body(t, pooled):
        d2 = d2_ref[...]
        mval = jnp.min(d2, axis=1, keepdims=True)
        cand = jnp.where(d2 == mval, iota, n)
        j = jnp.min(cand, axis=1, keepdims=True)
        oh = iota == j
        g = jax.lax.dot_general(oh.astype(jnp.float32), fp_ref[0],
                                (((1,), (0,)), ((), ())),
                                preferred_element_type=jnp.float32)
        e = jnp.maximum(g - ct, 0.0)
        pooled = jnp.maximum(pooled, jnp.where(mval <= r2, e, -1e9))
        d2_ref[...] = jnp.where(oh, 1e30, d2)
        return pooled

    pooled = jax.lax.fori_loop(
        0, K_NN, body, jnp.full((tc, 64), -1e9, jnp.float32))
    gate = jax.nn.sigmoid(
        jax.lax.dot_general(pooled, wg_ref[...], (((1,), (0,)), ((), ())),
                            preferred_element_type=jnp.float32) + bg_ref[...])
    out_ref[0] = pooled * gate


def _local_agg(coords, cpad, feats, cidx, wr, br, we, be, wg, bg, rmin, rmax):
    B, N, _ = coords.shape
    m = cidx.shape[1]
    TC = 256
    centers = jnp.take_along_axis(coords, cidx[..., None], axis=1)
    cfeat = jnp.take_along_axis(feats, cidx[..., None], axis=1)
    radius = rmin + jax.nn.sigmoid(cfeat @ wr + br) * (rmax - rmin)
    r2 = radius ** 2
    wf, wp = we[:64], we[64:67]
    fp = feats @ wf + coords @ wp + be
    cterm = centers @ wp
    c8 = jnp.concatenate([centers, r2, jnp.zeros((B, m, 4), jnp.float32)],
                         axis=-1)
    out = pl.pallas_call(
        _agg_body,
        grid=(B, m // TC),
        in_specs=[
            pl.BlockSpec((1, TC, 8), lambda b, t: (b, t, 0)),
            pl.BlockSpec((1, TC, 64), lambda b, t: (b, t, 0)),
            pl.BlockSpec((1, 8, N), lambda b, t: (b, 0, 0)),
            pl.BlockSpec((1, N, 64), lambda b, t: (b, 0, 0)),
            pl.BlockSpec((64, 64), lambda b, t: (0, 0)),
            pl.BlockSpec((1, 64), lambda b, t: (0, 0)),
        ],
        out_specs=pl.BlockSpec((1, TC, 64), lambda b, t: (b, t, 0)),
        out_shape=jax.ShapeDtypeStruct((B, m, 64), jnp.float32),
        scratch_shapes=[pltpu.VMEM((TC, N), jnp.float32)],
    )(c8, cterm, cpad, fp, wg, bg.reshape(1, 64))
    return out


def _channel_ccc(h, w1, b1, w2, b2):
    desc = jnp.mean(h, axis=1)
    a = jax.nn.sigmoid(jax.nn.relu(desc @ w1 + b1) @ w2 + b2)
    return h * a[:, None, :]


def _linear_gva(h, wq, wk, wv):
    q = h @ wq
    k = h @ wk
    v = h @ wv
    attn = jax.nn.softmax(k, axis=1)
    g = jnp.sum(attn * v, axis=1, keepdims=True)
    return h + jax.nn.sigmoid(q) * g


def _nearest_up(h, n):
    m = h.shape[1]
    idx = (jnp.arange(n) * m) // n
    return h[:, idx, :]


def kernel(x, enc_w0, enc_b0, enc_w1, enc_b1,
           f1_wr, f1_br, f1_we, f1_be, f1_wg, f1_bg,
           f2_wr, f2_br, f2_we, f2_be, f2_wg, f2_bg,
           aux_w0, aux_b0, aux_w1, aux_b1,
           ccc_w1, ccc_b1, ccc_w2, ccc_b2,
           gva_wq, gva_wk, gva_wv,
           up_w0, up_b0, up_w1, up_b1,
           cls_w0, cls_b0, cls_w1, cls_b1):
    B, N, _ = x.shape
    coords = x[..., :3]
    m1 = max(1, N // 2)
    m2 = max(1, N // 4)

    cpad = jnp.concatenate(
        [coords.transpose(0, 2, 1),
         jnp.zeros((B, 5, N), jnp.float32)], axis=1)
    idx1 = _fps_pallas(cpad, m1)
    idx2 = idx1[:, :m2]

    feats = jax.nn.relu(x @ enc_w0 + enc_b0) @ enc_w1 + enc_b1
    out1 = _local_agg(coords, cpad, feats, idx1, f1_wr, f1_br, f1_we, f1_be,
                      f1_wg, f1_bg, 0.02, 0.15)
    out2 = _local_agg(coords, cpad, feats, idx2, f2_wr, f2_br, f2_we, f2_be,
                      f2_wg, f2_bg, 0.05, 0.3)
    out1 = _channel_ccc(out1, ccc_w1, ccc_b1, ccc_w2, ccc_b2)
    out2 = _channel_ccc(out2, ccc_w1, ccc_b1, ccc_w2, ccc_b2)
    out1 = _linear_gva(out1, gva_wq, gva_wk, gva_wv)
    out2 = _linear_gva(out2, gva_wq, gva_wk, gva_wv)
    out1_up = _nearest_up(out1, N)
    out2_up = _nearest_up(out2, N)
    fused = jnp.concatenate([out1_up, out2_up], axis=-1)
    fused = jax.nn.relu(fused @ up_w0 + up_b0) @ up_w1 + up_b1
    logits = jax.nn.relu(fused @ cls_w0 + cls_b0) @ cls_w1 + cls_b1
    return logits


# speedup vs baseline: 12.6967x; 1.7886x over previous
import functools

import jax
import jax.numpy as jnp
from jax.experimental import pallas as pl
from jax.experimental.pallas import tpu as pltpu

K_NN = 32


def _red2(v, fn):
    return fn(fn(v, axis=0, keepdims=True), axis=1, keepdims=True)


def _fps_body(m, c_ref, o_ref):
    B = c_ref.shape[0]
    cols = c_ref.shape[3]
    n = 8 * cols
    mc = m // 8
    planes = [[c_ref[b, k] for k in range(3)] for b in range(B)]
    fi = (jax.lax.broadcasted_iota(jnp.int32, (8, cols), 0) * cols
          + jax.lax.broadcasted_iota(jnp.int32, (8, cols), 1))
    fim = (jax.lax.broadcasted_iota(jnp.int32, (8, mc), 0) * mc
           + jax.lax.broadcasted_iota(jnp.int32, (8, mc), 1))

    def body(i, carry):
        dists, fars, idxss = carry
        new_d, new_f, new_i = [], [], []
        for b in range(B):
            dist, far, idxs = dists[b], fars[b], idxss[b]
            idxs = jnp.where(fim == i, far, idxs)
            oh = fi == far
            xb, yb, zb = planes[b]
            cx = _red2(jnp.where(oh, xb, 0.0), jnp.sum)
            cy = _red2(jnp.where(oh, yb, 0.0), jnp.sum)
            cz = _red2(jnp.where(oh, zb, 0.0), jnp.sum)
            dx, dy, dz = xb - cx, yb - cy, zb - cz
            d = (dx * dx + dy * dy) + dz * dz
            dist = jnp.minimum(dist, d)
            mx = _red2(dist, jnp.max)
            sel = jnp.where(dist == mx, fi, n)
            far2 = _red2(sel, jnp.min)
            new_d.append(dist)
            new_f.append(far2)
            new_i.append(idxs)
        return tuple(new_d), tuple(new_f), tuple(new_i)

    dist0 = tuple(jnp.full((8, cols), 1e10, jnp.float32) for _ in range(B))
    far0 = tuple(jnp.zeros((1, 1), jnp.int32) for _ in range(B))
    idxs0 = tuple(jnp.zeros((8, mc), jnp.int32) for _ in range(B))
    _, _, idxss = jax.lax.fori_loop(0, m, body, (dist0, far0, idxs0))
    for b in range(B):
        o_ref[b] = idxss[b]


def _fps_pallas(coords, m):
    B, N, _ = coords.shape
    planes = coords.transpose(0, 2, 1).reshape(B, 3, 8, N // 8)
    out = pl.pallas_call(
        functools.partial(_fps_body, m),
        in_specs=[pl.BlockSpec((B, 3, 8, N // 8), lambda: (0, 0, 0, 0))],
        out_specs=pl.BlockSpec((B, 8, m // 8), lambda: (0, 0, 0)),
        out_shape=jax.ShapeDtypeStruct((B, 8, m // 8), jnp.int32),
    )(planes)
    return out.reshape(B, m)


def _agg_body(c8_ref, ct_ref, coords_ref, fp_ref, wg_ref, bg_ref,
              out_ref, d2_ref):
    tc = c8_ref.shape[1]
    n = coords_ref.shape[2]
    c8 = c8_ref[0]
    cx, cy, cz = c8[:, 0:1], c8[:, 1:2], c8[:, 2:3]
    r2 = c8[:, 3:4]
    px = coords_ref[0, 0:1, :]
    py = coords_ref[0, 1:2, :]
    pz = coords_ref[0, 2:3, :]
    dx, dy, dz = cx - px, cy - py, cz - pz
    d2_ref[...] = (dx * dx + dy * dy) + dz * dz
    ct = ct_ref[0]
    iota = jax.lax.broadcasted_iota(jnp.int32, (1, n), 1)

    def body(t, pooled):
        d2 = d2_ref[...]
        mval = jnp.min(d2, axis=1, keepdims=True)
        cand = jnp.where(d2 == mval, iota, n)
        j = jnp.min(cand, axis=1, keepdims=True)
        oh = iota == j
        g = jax.lax.dot_general(oh.astype(jnp.float32), fp_ref[0],
                                (((1,), (0,)), ((), ())),
                                preferred_element_type=jnp.float32)
        e = jnp.maximum(g - ct, 0.0)
        pooled = jnp.maximum(pooled, jnp.where(mval <= r2, e, -1e9))
        d2_ref[...] = jnp.where(oh, 1e30, d2)
        return pooled

    pooled = jax.lax.fori_loop(
        0, K_NN, body, jnp.full((tc, 64), -1e9, jnp.float32))
    gate = jax.nn.sigmoid(
        jax.lax.dot_general(pooled, wg_ref[...], (((1,), (0,)), ((), ())),
                            preferred_element_type=jnp.float32) + bg_ref[...])
    out_ref[0] = pooled * gate


def _local_agg(coords, cpad, feats, cidx, wr, br, we, be, wg, bg, rmin, rmax):
    B, N, _ = coords.shape
    m = cidx.shape[1]
    TC = 256
    centers = jnp.take_along_axis(coords, cidx[..., None], axis=1)
    cfeat = jnp.take_along_axis(feats, cidx[..., None], axis=1)
    radius = rmin + jax.nn.sigmoid(cfeat @ wr + br) * (rmax - rmin)
    r2 = radius ** 2
    wf, wp = we[:64], we[64:67]
    fp = feats @ wf + coords @ wp + be
    cterm = centers @ wp
    c8 = jnp.concatenate([centers, r2, jnp.zeros((B, m, 4), jnp.float32)],
                         axis=-1)
    out = pl.pallas_call(
        _agg_body,
        grid=(B, m // TC),
        in_specs=[
            pl.BlockSpec((1, TC, 8), lambda b, t: (b, t, 0)),
            pl.BlockSpec((1, TC, 64), lambda b, t: (b, t, 0)),
            pl.BlockSpec((1, 8, N), lambda b, t: (b, 0, 0)),
            pl.BlockSpec((1, N, 64), lambda b, t: (b, 0, 0)),
            pl.BlockSpec((64, 64), lambda b, t: (0, 0)),
            pl.BlockSpec((1, 64), lambda b, t: (0, 0)),
        ],
        out_specs=pl.BlockSpec((1, TC, 64), lambda b, t: (b, t, 0)),
        out_shape=jax.ShapeDtypeStruct((B, m, 64), jnp.float32),
        scratch_shapes=[pltpu.VMEM((TC, N), jnp.float32)],
    )(c8, cterm, cpad, fp, wg, bg.reshape(1, 64))
    return out


def _channel_ccc(h, w1, b1, w2, b2):
    desc = jnp.mean(h, axis=1)
    a = jax.nn.sigmoid(jax.nn.relu(desc @ w1 + b1) @ w2 + b2)
    return h * a[:, None, :]


def _linear_gva(h, wq, wk, wv):
    q = h @ wq
    k = h @ wk
    v = h @ wv
    attn = jax.nn.softmax(k, axis=1)
    g = jnp.sum(attn * v, axis=1, keepdims=True)
    return h + jax.nn.sigmoid(q) * g


def _nearest_up(h, n):
    m = h.shape[1]
    idx = (jnp.arange(n) * m) // n
    return h[:, idx, :]


def kernel(x, enc_w0, enc_b0, enc_w1, enc_b1,
           f1_wr, f1_br, f1_we, f1_be, f1_wg, f1_bg,
           f2_wr, f2_br, f2_we, f2_be, f2_wg, f2_bg,
           aux_w0, aux_b0, aux_w1, aux_b1,
           ccc_w1, ccc_b1, ccc_w2, ccc_b2,
           gva_wq, gva_wk, gva_wv,
           up_w0, up_b0, up_w1, up_b1,
           cls_w0, cls_b0, cls_w1, cls_b1):
    B, N, _ = x.shape
    coords = x[..., :3]
    m1 = max(1, N // 2)
    m2 = max(1, N // 4)

    cpad = jnp.concatenate(
        [coords.transpose(0, 2, 1),
         jnp.zeros((B, 5, N), jnp.float32)], axis=1)
    idx1 = _fps_pallas(coords, m1)
    idx2 = idx1[:, :m2]

    feats = jax.nn.relu(x @ enc_w0 + enc_b0) @ enc_w1 + enc_b1
    out1 = _local_agg(coords, cpad, feats, idx1, f1_wr, f1_br, f1_we, f1_be,
                      f1_wg, f1_bg, 0.02, 0.15)
    out2 = _local_agg(coords, cpad, feats, idx2, f2_wr, f2_br, f2_we, f2_be,
                      f2_wg, f2_bg, 0.05, 0.3)
    out1 = _channel_ccc(out1, ccc_w1, ccc_b1, ccc_w2, ccc_b2)
    out2 = _channel_ccc(out2, ccc_w1, ccc_b1, ccc_w2, ccc_b2)
    out1 = _linear_gva(out1, gva_wq, gva_wk, gva_wv)
    out2 = _linear_gva(out2, gva_wq, gva_wk, gva_wv)
    out1_up = _nearest_up(out1, N)
    out2_up = _nearest_up(out2, N)
    fused = jnp.concatenate([out1_up, out2_up], axis=-1)
    fused = jax.nn.relu(fused @ up_w0 + up_b0) @ up_w1 + up_b1
    logits = jax.nn.relu(fused @ cls_w0 + cls_b0) @ cls_w1 + cls_b1
    return logits


# speedup vs baseline: 26.6975x; 2.1027x over previous
import functools

import jax
import jax.numpy as jnp
from jax.experimental import pallas as pl
from jax.experimental.pallas import tpu as pltpu

K_NN = 32


def _red2(v, fn):
    return fn(fn(v, axis=0, keepdims=True), axis=1, keepdims=True)


def _fps_body(m, c_ref, o_ref):
    B = c_ref.shape[0]
    cols = c_ref.shape[3]
    n = 8 * cols
    mc = m // 8
    planes = [[c_ref[b, k] for k in range(3)] for b in range(B)]
    fi = (jax.lax.broadcasted_iota(jnp.int32, (8, cols), 0) * cols
          + jax.lax.broadcasted_iota(jnp.int32, (8, cols), 1))
    fim = (jax.lax.broadcasted_iota(jnp.int32, (8, mc), 0) * mc
           + jax.lax.broadcasted_iota(jnp.int32, (8, mc), 1))

    def body(i, carry):
        dists, fars, idxss = carry
        new_d, new_f, new_i = [], [], []
        for b in range(B):
            dist, far, idxs = dists[b], fars[b], idxss[b]
            idxs = jnp.where(fim == i, far, idxs)
            oh = fi == far
            xb, yb, zb = planes[b]
            cx = _red2(jnp.where(oh, xb, 0.0), jnp.sum)
            cy = _red2(jnp.where(oh, yb, 0.0), jnp.sum)
            cz = _red2(jnp.where(oh, zb, 0.0), jnp.sum)
            dx, dy, dz = xb - cx, yb - cy, zb - cz
            d = (dx * dx + dy * dy) + dz * dz
            dist = jnp.minimum(dist, d)
            mx = _red2(dist, jnp.max)
            sel = jnp.where(dist == mx, fi, n)
            far2 = _red2(sel, jnp.min)
            new_d.append(dist)
            new_f.append(far2)
            new_i.append(idxs)
        return tuple(new_d), tuple(new_f), tuple(new_i)

    dist0 = tuple(jnp.full((8, cols), 1e10, jnp.float32) for _ in range(B))
    far0 = tuple(jnp.zeros((1, 1), jnp.int32) for _ in range(B))
    idxs0 = tuple(jnp.zeros((8, mc), jnp.int32) for _ in range(B))
    _, _, idxss = jax.lax.fori_loop(0, m, body, (dist0, far0, idxs0))
    for b in range(B):
        o_ref[b] = idxss[b]


def _fps_pallas(coords, m):
    B, N, _ = coords.shape
    planes = coords.transpose(0, 2, 1).reshape(B, 3, 8, N // 8)
    out = pl.pallas_call(
        functools.partial(_fps_body, m),
        in_specs=[pl.BlockSpec((B, 3, 8, N // 8), lambda: (0, 0, 0, 0))],
        out_specs=pl.BlockSpec((B, 8, m // 8), lambda: (0, 0, 0)),
        out_shape=jax.ShapeDtypeStruct((B, 8, m // 8), jnp.int32),
    )(planes)
    return out.reshape(B, m)


def _agg_body(c8_ref, ct_ref, coords_ref, fp_ref, wg_ref, bg_ref,
              out_ref, d2_ref):
    tc = c8_ref.shape[1]
    n = coords_ref.shape[2]
    c8 = c8_ref[0]
    cx, cy, cz = c8[:, 0:1], c8[:, 1:2], c8[:, 2:3]
    r2 = c8[:, 3:4]
    px = coords_ref[0, 0:1, :]
    py = coords_ref[0, 1:2, :]
    pz = coords_ref[0, 2:3, :]
    dx, dy, dz = cx - px, cy - py, cz - pz
    d2_ref[...] = (dx * dx + dy * dy) + dz * dz
    ct = ct_ref[0]
    iota = jax.lax.broadcasted_iota(jnp.int32, (1, n), 1)

    def cond(carry):
        t, _, go = carry
        return jnp.logical_and(t < K_NN, go)

    def body(carry):
        t, pooled, _ = carry
        d2 = d2_ref[...]
        mval = jnp.min(d2, axis=1, keepdims=True)
        alive = mval <= r2
        cand = jnp.where(d2 == mval, iota, n)
        j = jnp.min(cand, axis=1, keepdims=True)
        oh = iota == j
        g = jax.lax.dot_general(oh.astype(jnp.float32), fp_ref[0],
                                (((1,), (0,)), ((), ())),
                                preferred_element_type=jnp.float32)
        e = jnp.maximum(g - ct, 0.0)
        pooled = jnp.maximum(pooled, jnp.where(alive, e, -1e9))
        d2_ref[...] = jnp.where(oh, 1e30, d2)
        return t + 1, pooled, jnp.any(alive)

    _, pooled, _ = jax.lax.while_loop(
        cond, body,
        (jnp.int32(0), jnp.full((tc, 64), -1e9, jnp.float32),
         jnp.bool_(True)))
    gate = jax.nn.sigmoid(
        jax.lax.dot_general(pooled, wg_ref[...], (((1,), (0,)), ((), ())),
                            preferred_element_type=jnp.float32) + bg_ref[...])
    out_ref[0] = pooled * gate


def _local_agg(coords, cpad, feats, cidx, wr, br, we, be, wg, bg, rmin, rmax):
    B, N, _ = coords.shape
    m = cidx.shape[1]
    TC = 256
    centers = jnp.take_along_axis(coords, cidx[..., None], axis=1)
    cfeat = jnp.take_along_axis(feats, cidx[..., None], axis=1)
    radius = rmin + jax.nn.sigmoid(cfeat @ wr + br) * (rmax - rmin)
    r2 = radius ** 2
    wf, wp = we[:64], we[64:67]
    fp = feats @ wf + coords @ wp + be
    cterm = centers @ wp
    c8 = jnp.concatenate([centers, r2, jnp.zeros((B, m, 4), jnp.float32)],
                         axis=-1)
    out = pl.pallas_call(
        _agg_body,
        grid=(B, m // TC),
        in_specs=[
            pl.BlockSpec((1, TC, 8), lambda b, t: (b, t, 0)),
            pl.BlockSpec((1, TC, 64), lambda b, t: (b, t, 0)),
            pl.BlockSpec((1, 8, N), lambda b, t: (b, 0, 0)),
            pl.BlockSpec((1, N, 64), lambda b, t: (b, 0, 0)),
            pl.BlockSpec((64, 64), lambda b, t: (0, 0)),
            pl.BlockSpec((1, 64), lambda b, t: (0, 0)),
        ],
        out_specs=pl.BlockSpec((1, TC, 64), lambda b, t: (b, t, 0)),
        out_shape=jax.ShapeDtypeStruct((B, m, 64), jnp.float32),
        scratch_shapes=[pltpu.VMEM((TC, N), jnp.float32)],
    )(c8, cterm, cpad, fp, wg, bg.reshape(1, 64))
    return out


def _channel_ccc(h, w1, b1, w2, b2):
    desc = jnp.mean(h, axis=1)
    a = jax.nn.sigmoid(jax.nn.relu(desc @ w1 + b1) @ w2 + b2)
    return h * a[:, None, :]


def _linear_gva(h, wq, wk, wv):
    q = h @ wq
    k = h @ wk
    v = h @ wv
    attn = jax.nn.softmax(k, axis=1)
    g = jnp.sum(attn * v, axis=1, keepdims=True)
    return h + jax.nn.sigmoid(q) * g


def _nearest_up(h, n):
    m = h.shape[1]
    idx = (jnp.arange(n) * m) // n
    return h[:, idx, :]


def kernel(x, enc_w0, enc_b0, enc_w1, enc_b1,
           f1_wr, f1_br, f1_we, f1_be, f1_wg, f1_bg,
           f2_wr, f2_br, f2_we, f2_be, f2_wg, f2_bg,
           aux_w0, aux_b0, aux_w1, aux_b1,
           ccc_w1, ccc_b1, ccc_w2, ccc_b2,
           gva_wq, gva_wk, gva_wv,
           up_w0, up_b0, up_w1, up_b1,
           cls_w0, cls_b0, cls_w1, cls_b1):
    B, N, _ = x.shape
    coords = x[..., :3]
    m1 = max(1, N // 2)
    m2 = max(1, N // 4)

    cpad = jnp.concatenate(
        [coords.transpose(0, 2, 1),
         jnp.zeros((B, 5, N), jnp.float32)], axis=1)
    idx1 = _fps_pallas(coords, m1)
    idx2 = idx1[:, :m2]

    feats = jax.nn.relu(x @ enc_w0 + enc_b0) @ enc_w1 + enc_b1
    out1 = _local_agg(coords, cpad, feats, idx1, f1_wr, f1_br, f1_we, f1_be,
                      f1_wg, f1_bg, 0.02, 0.15)
    out2 = _local_agg(coords, cpad, feats, idx2, f2_wr, f2_br, f2_we, f2_be,
                      f2_wg, f2_bg, 0.05, 0.3)
    out1 = _channel_ccc(out1, ccc_w1, ccc_b1, ccc_w2, ccc_b2)
    out2 = _channel_ccc(out2, ccc_w1, ccc_b1, ccc_w2, ccc_b2)
    out1 = _linear_gva(out1, gva_wq, gva_wk, gva_wv)
    out2 = _linear_gva(out2, gva_wq, gva_wk, gva_wv)
    out1_up = _nearest_up(out1, N)
    out2_up = _nearest_up(out2, N)
    fused = jnp.concatenate([out1_up, out2_up], axis=-1)
    fused = jax.nn.relu(fused @ up_w0 + up_b0) @ up_w1 + up_b1
    logits = jax.nn.relu(fused @ cls_w0 + cls_b0) @ cls_w1 + cls_b1
    return logits
